# baseline (speedup 1.0000x reference)
import jax
import jax.numpy as jnp
from jax.experimental import pallas as pl

_TEMPERATURE = 0.07
_NEG_INF = -1e30


def _loss_kernel(tok_ref, lc_ref, lr_ref, out_ref):
    lc = lc_ref[:, :]
    lr = lr_ref[:, :]
    B = lc.shape[0]
    same_f = jnp.where(lc == lr, jnp.float32(1.0), jnp.float32(0.0))
    row = jax.lax.broadcasted_iota(jnp.int32, (B, B), 0)
    col = jax.lax.broadcasted_iota(jnp.int32, (B, B), 1)
    eye = row == col
    pos_count = jnp.sum(same_f, axis=1, keepdims=True) - 1.0
    valid = pos_count > 0.0
    inv_cnt = 1.0 / jnp.maximum(pos_count, 1.0)

    total = jnp.float32(0.0)
    scale = jnp.float32(1.0 / _TEMPERATURE) ** 0.5
    for j in range(tok_ref.shape[0]):
        tj = tok_ref[j] * scale
        S = jax.lax.dot_general(
            tj, tj, (((1,), (1,)), ((), ())),
            preferred_element_type=jnp.float32)
        Sm = jnp.where(eye, jnp.float32(_NEG_INF), S)
        m = jnp.max(Sm, axis=1, keepdims=True)
        lse = m + jnp.log(jnp.sum(jnp.exp(Sm - m), axis=1, keepdims=True))
        g = jax.lax.dot_general(
            same_f, tj, (((1,), (0,)), ((), ())),
            preferred_element_type=jnp.float32)
        pos_dot = jnp.sum(tj * g, axis=1, keepdims=True)
        self_sq = jnp.sum(tj * tj, axis=1, keepdims=True)
        pos_sum = (pos_dot - self_sq) - pos_count * lse
        total = total + jnp.sum(jnp.where(valid, pos_sum * inv_cnt, 0.0))

    nvalid = jnp.sum(jnp.where(valid, jnp.float32(1.0), jnp.float32(0.0)))
    m_f = jnp.float32(tok_ref.shape[0])
    out_ref[:, :] = (total / (-m_f * nvalid)).reshape(1, 1)


def kernel(tokens, labels):
    if tokens.ndim == 2:
        tokens = tokens[:, None, :]
    tokens = jnp.transpose(tokens, (1, 0, 2)).astype(jnp.float32)
    labels = labels.astype(jnp.int32)
    B = tokens.shape[1]
    lc = labels.reshape(B, 1)
    lr = labels.reshape(1, B)
    out = pl.pallas_call(
        _loss_kernel,
        out_shape=jax.ShapeDtypeStruct((1, 1), jnp.float32),
    )(tokens, lc, lr)
    return out[0, 0]
